# Initial kernel scaffold; baseline (speedup 1.0000x reference)
import jax
import jax.numpy as jnp
from jax.experimental import pallas as pl


def kernel(node_inds, adj_mat_inds, init_hydrogens, init_charge, init_is_in_ring, init_is_aromatic, init_chirality, n_table, e_table, h_table, charge_table, ring_table, arom_table, chir_table):
    raise NotImplementedError("write your pallas kernel here")


# speedup vs baseline: 2.9636x; 2.9636x over previous
import jax
import jax.numpy as jnp
from jax.experimental import pallas as pl
from jax.experimental.pallas import tpu as pltpu

_B = 64
_N = 40
_H = 256
_K = 2
_E = _H * _K
_NN = _N * _N
_NODE_SIZES = (22, 6, 6, 3, 3, 5)
_NT = sum(_NODE_SIZES)


_TT = (((0,), (0,)), ((), ()))


def _body(nidx_ref, aidx_ref, ntab_ref, etab_ref, node_out_ref, edge_out_ref):
    iota_nt = jax.lax.broadcasted_iota(jnp.int32, (_NT, _N), 0)
    cnt_t = jnp.zeros((_NT, _N), jnp.float32)
    off = 0
    for t, sz in enumerate(_NODE_SIZES):
        idx_row = nidx_ref[0, t:t + 1, :]
        cnt_t += (iota_nt == (idx_row + off)).astype(jnp.float32)
        off += sz
    node = jax.lax.dot_general(cnt_t, ntab_ref[...], _TT,
                               preferred_element_type=jnp.float32)
    node_out_ref[...] = node[None]

    a_row = aidx_ref[0]
    iota_e = jax.lax.broadcasted_iota(jnp.int32, (8, _NN), 0)
    oh_t = (iota_e == a_row).astype(jnp.float32)
    etab8 = jnp.concatenate(
        [etab_ref[...], jnp.zeros((1, _E), jnp.float32)], axis=0)
    edge = jax.lax.dot_general(oh_t, etab8, _TT,
                               preferred_element_type=jnp.float32)
    edge_out_ref[...] = edge[None]


def kernel(node_inds, adj_mat_inds, init_hydrogens, init_charge,
           init_is_in_ring, init_is_aromatic, init_chirality,
           n_table, e_table, h_table, charge_table, ring_table,
           arom_table, chir_table):
    idx6 = jnp.stack([node_inds, init_hydrogens, init_charge,
                      init_is_in_ring, init_is_aromatic, init_chirality], 1)
    idx6 = idx6.astype(jnp.int32)
    adj_f = adj_mat_inds.reshape(_B, 1, _NN).astype(jnp.int32)
    ntab = jnp.concatenate([n_table, h_table, charge_table, ring_table,
                            arom_table, chir_table], 0)

    node_out, edge_out = pl.pallas_call(
        _body,
        grid=(_B,),
        in_specs=[
            pl.BlockSpec((1, 6, _N), lambda b: (b, 0, 0)),
            pl.BlockSpec((1, 1, _NN), lambda b: (b, 0, 0)),
            pl.BlockSpec((_NT, _H), lambda b: (0, 0)),
            pl.BlockSpec((7, _E), lambda b: (0, 0)),
        ],
        out_specs=[
            pl.BlockSpec((1, _N, _H), lambda b: (b, 0, 0)),
            pl.BlockSpec((1, _NN, _E), lambda b: (b, 0, 0)),
        ],
        out_shape=[
            jax.ShapeDtypeStruct((_B, _N, _H), jnp.float32),
            jax.ShapeDtypeStruct((_B, _NN, _E), jnp.float32),
        ],
    )(idx6, adj_f, ntab, e_table)

    return node_out, edge_out.reshape(_B, _N, _N, _H, _K)


# speedup vs baseline: 22.1172x; 7.4630x over previous
import jax
import jax.numpy as jnp
from jax.experimental import pallas as pl

_B = 64
_N = 40
_H = 256
_K = 2
_E = _H * _K
_NN = _N * _N
_S = _NN * 4
_NODE_SIZES = (22, 6, 6, 3, 3, 5)
_NT = sum(_NODE_SIZES)

_TT = (((0,), (0,)), ((), ()))


def _body(nidx_ref, comb_ref, ntab_ref, wtab_ref, node_out_ref, edge_out_ref):
    iota_nt = jax.lax.broadcasted_iota(jnp.int32, (_NT, _N), 0)
    cnt_t = jnp.zeros((_NT, _N), jnp.float32)
    off = 0
    for t, sz in enumerate(_NODE_SIZES):
        idx_row = nidx_ref[0, t:t + 1, :]
        cnt_t += (iota_nt == (idx_row + off)).astype(jnp.float32)
        off += sz
    node = jax.lax.dot_general(cnt_t, ntab_ref[...], _TT,
                               preferred_element_type=jnp.float32)
    node_out_ref[...] = node[None]

    comb = comb_ref[0]
    iota_e = jax.lax.broadcasted_iota(jnp.int32, (32, _S), 0)
    oh = (iota_e == comb).astype(jnp.float32)
    edge = jax.lax.dot_general(oh, wtab_ref[...], _TT,
                               preferred_element_type=jnp.float32)
    edge_out_ref[...] = edge


def kernel(node_inds, adj_mat_inds, init_hydrogens, init_charge,
           init_is_in_ring, init_is_aromatic, init_chirality,
           n_table, e_table, h_table, charge_table, ring_table,
           arom_table, chir_table):
    idx6 = jnp.stack([node_inds, init_hydrogens, init_charge,
                      init_is_in_ring, init_is_aromatic, init_chirality], 1)
    idx6 = idx6.astype(jnp.int32)
    adj = adj_mat_inds.astype(jnp.int32).reshape(_B, _NN, 1)
    comb = (adj * 4 + jnp.arange(4, dtype=jnp.int32)).reshape(_B, 1, _S)
    ntab = jnp.concatenate([n_table, h_table, charge_table, ring_table,
                            arom_table, chir_table], 0)
    wtab = e_table.reshape(7, 2, 128, 2).transpose(0, 1, 3, 2).reshape(28, 128)
    wtab = jnp.concatenate([wtab, jnp.zeros((4, 128), jnp.float32)], 0)

    node_out, edge_out = pl.pallas_call(
        _body,
        grid=(_B,),
        in_specs=[
            pl.BlockSpec((1, 6, _N), lambda b: (b, 0, 0)),
            pl.BlockSpec((1, 1, _S), lambda b: (b, 0, 0)),
            pl.BlockSpec((_NT, _H), lambda b: (0, 0)),
            pl.BlockSpec((32, 128), lambda b: (0, 0)),
        ],
        out_specs=[
            pl.BlockSpec((1, _N, _H), lambda b: (b, 0, 0)),
            pl.BlockSpec((_S, 128), lambda b: (b, 0)),
        ],
        out_shape=[
            jax.ShapeDtypeStruct((_B, _N, _H), jnp.float32),
            jax.ShapeDtypeStruct((_B * _S, 128), jnp.float32),
        ],
    )(idx6, comb, ntab, wtab)

    edge5 = (edge_out.reshape(_B, _N, _N, 2, _K, 128)
             .transpose(0, 1, 2, 3, 5, 4)
             .reshape(_B, _N, _N, _H, _K))
    return node_out, edge5
